# baseline (speedup 1.0000x reference)
import functools

import jax
import jax.numpy as jnp
from jax import lax
from jax.experimental import pallas as pl
from jax.experimental.pallas import tpu as pltpu
from jax.experimental.pallas import tpu_sc as plsc

_N = 1000000
_D = 64
_B = 16384
_R = 10000
_NSTEP = (_N // 2) // _R
_F = _R * 2 * _D


def _sigma_body(ue_ref, uo_ref, e_ref, sig_ref, g_acc, te_acc,
                to_acc):
    i = pl.program_id(0)

    @pl.when(i == 0)
    def _init():
        g_acc[...] = jnp.zeros_like(g_acc)
        te_acc[...] = jnp.zeros_like(te_acc)
        to_acc[...] = jnp.zeros_like(to_acc)

    eb = e_ref[...]
    g_acc[...] += lax.dot_general(
        eb, eb, (((0,), (0,)), ((), ())), preferred_element_type=jnp.float32)
    te_acc[...] += lax.dot_general(
        ue_ref[0].astype(jnp.bfloat16), eb, (((1,), (0,)), ((), ())),
        preferred_element_type=jnp.float32)
    to_acc[...] += lax.dot_general(
        uo_ref[0].astype(jnp.bfloat16), eb, (((1,), (0,)), ((), ())),
        preferred_element_type=jnp.float32)

    @pl.when(i == pl.num_programs(0) - 1)
    def _finish():
        te = te_acc[...]
        to = to_acc[...]
        t = te[:, :_D] + to[:, _D:]
        v = t * lax.rsqrt(jnp.maximum(jnp.sum(t * t), 1e-12))
        g = g_acc[...]
        gf = g[:_D, :_D] + g[_D:, _D:]
        gv = lax.dot_general(
            v, gf, (((1,), (0,)), ((), ())),
            preferred_element_type=jnp.float32)
        s2 = jnp.maximum(jnp.sum(gv * v), 1e-12)
        sig_ref[...] = lax.rsqrt(s2) * jnp.ones_like(sig_ref)


def _sigma_pass(e2m, ue, uo):
    return pl.pallas_call(
        _sigma_body,
        grid=(_NSTEP,),
        in_specs=[
            pl.BlockSpec((1, 1, _R), lambda i: (i, 0, 0)),
            pl.BlockSpec((1, 1, _R), lambda i: (i, 0, 0)),
            pl.BlockSpec((_R, 2 * _D), lambda i: (i, 0)),
        ],
        out_specs=pl.BlockSpec((1, 1), lambda i: (0, 0)),
        out_shape=jax.ShapeDtypeStruct((1, 1), jnp.float32),
        scratch_shapes=[
            pltpu.VMEM((2 * _D, 2 * _D), jnp.float32),
            pltpu.VMEM((1, 2 * _D), jnp.float32),
            pltpu.VMEM((1, 2 * _D), jnp.float32),
        ],
    )(ue, uo, e2m)


def _scale_body(s_ref, x_ref, o_ref):
    o_ref[...] = x_ref[...] * s_ref[0, 0]


def _scale(inv_sigma, rows):
    return pl.pallas_call(
        _scale_body,
        in_specs=[
            pl.BlockSpec(memory_space=pltpu.SMEM),
            pl.BlockSpec((_B, _D), lambda: (0, 0)),
        ],
        out_specs=pl.BlockSpec((_B, _D), lambda: (0, 0)),
        out_shape=jax.ShapeDtypeStruct((_B, _D), jnp.float32),
    )(inv_sigma, rows)


@functools.lru_cache(maxsize=1)
def _make_gather():
    info = plsc.get_sparse_core_info()
    nw = info.num_cores * info.num_subcores
    bpw = _B // nw
    mesh = plsc.VectorSubcoreMesh(core_axis_name="c", subcore_axis_name="s")

    @functools.partial(
        pl.kernel, mesh=mesh,
        out_type=jax.ShapeDtypeStruct((_B, _D), jnp.float32),
        scratch_types=[
            pltpu.VMEM((bpw,), jnp.int32),
            pltpu.VMEM((bpw, _D), jnp.float32),
            pltpu.SemaphoreType.DMA,
        ],
        compiler_params=pltpu.CompilerParams(use_tc_tiling_on_sc=False),
    )
    def gather(table_hbm, idx_hbm, out_hbm, idx_v, rows_v, sem):
        wid = lax.axis_index("s") * info.num_cores + lax.axis_index("c")
        base = wid * bpw
        pltpu.sync_copy(idx_hbm.at[pl.ds(base, bpw)], idx_v)
        pltpu.async_copy(table_hbm.at[idx_v], rows_v, sem).wait()
        pltpu.sync_copy(rows_v, out_hbm.at[pl.ds(base, bpw)])

    return gather


def kernel(indices, embeddings, u):
    idx = indices.astype(jnp.int32)
    e2mb = embeddings.astype(jnp.bfloat16).reshape(_N // 2, 2 * _D)
    u1 = u.reshape(_N)
    ue = u1[0::2].reshape(_NSTEP, 1, _R)
    uo = u1[1::2].reshape(_NSTEP, 1, _R)
    inv_sigma = _sigma_pass(e2mb, ue, uo)
    idxg = idx + jnp.int32(inv_sigma[0, 0] * 0.0)
    rows = _make_gather()(embeddings, idxg)
    return _scale(inv_sigma, rows)


# speedup vs baseline: 1.6185x; 1.6185x over previous
import functools

import jax
import jax.numpy as jnp
from jax import lax
from jax.experimental import pallas as pl
from jax.experimental.pallas import tpu as pltpu
from jax.experimental.pallas import tpu_sc as plsc

_N = 1000000
_D = 64
_B = 16384
_R = 10000
_NSTEP = (_N // 2) // _R
_F = _R * 2 * _D


def _sigma_body(ue_ref, uo_ref, e_ref, sig_ref, g_acc, te_acc,
                to_acc):
    i = pl.program_id(0)

    @pl.when(i == 0)
    def _init():
        g_acc[...] = jnp.zeros_like(g_acc)
        te_acc[...] = jnp.zeros_like(te_acc)
        to_acc[...] = jnp.zeros_like(to_acc)

    e2 = e_ref[...]
    eb = e2.astype(jnp.bfloat16)
    g_acc[...] += lax.dot_general(
        eb, eb, (((0,), (0,)), ((), ())), preferred_element_type=jnp.float32)
    te_acc[...] += lax.dot_general(
        ue_ref[0], e2, (((1,), (0,)), ((), ())),
        preferred_element_type=jnp.float32)
    to_acc[...] += lax.dot_general(
        uo_ref[0], e2, (((1,), (0,)), ((), ())),
        preferred_element_type=jnp.float32)

    @pl.when(i == pl.num_programs(0) - 1)
    def _finish():
        te = te_acc[...]
        to = to_acc[...]
        t = te[:, :_D] + to[:, _D:]
        v = t * lax.rsqrt(jnp.maximum(jnp.sum(t * t), 1e-12))
        g = g_acc[...]
        gf = g[:_D, :_D] + g[_D:, _D:]
        gv = lax.dot_general(
            v, gf, (((1,), (0,)), ((), ())),
            preferred_element_type=jnp.float32)
        s2 = jnp.maximum(jnp.sum(gv * v), 1e-12)
        sig_ref[...] = lax.rsqrt(s2) * jnp.ones_like(sig_ref)


def _sigma_pass(e2m, ue, uo):
    return pl.pallas_call(
        _sigma_body,
        grid=(_NSTEP,),
        in_specs=[
            pl.BlockSpec((1, 1, _R), lambda i: (i, 0, 0)),
            pl.BlockSpec((1, 1, _R), lambda i: (i, 0, 0)),
            pl.BlockSpec((_R, 2 * _D), lambda i: (i, 0)),
        ],
        out_specs=pl.BlockSpec((1, 1), lambda i: (0, 0)),
        out_shape=jax.ShapeDtypeStruct((1, 1), jnp.float32),
        scratch_shapes=[
            pltpu.VMEM((2 * _D, 2 * _D), jnp.float32),
            pltpu.VMEM((1, 2 * _D), jnp.float32),
            pltpu.VMEM((1, 2 * _D), jnp.float32),
        ],
    )(ue, uo, e2m)


def _scale_body(s_ref, p_ref, x_ref, o_ref):
    lo = x_ref[:, : _D]
    hi = x_ref[:, _D:]
    sel = jnp.where(p_ref[...] > 0.5, hi, lo)
    o_ref[...] = sel * s_ref[0, 0]


def _scale(inv_sigma, parity, rows2):
    return pl.pallas_call(
        _scale_body,
        in_specs=[
            pl.BlockSpec(memory_space=pltpu.SMEM),
            pl.BlockSpec((_B, 1), lambda: (0, 0)),
            pl.BlockSpec((_B, 2 * _D), lambda: (0, 0)),
        ],
        out_specs=pl.BlockSpec((_B, _D), lambda: (0, 0)),
        out_shape=jax.ShapeDtypeStruct((_B, _D), jnp.float32),
    )(inv_sigma, parity, rows2)


@functools.lru_cache(maxsize=1)
def _make_gather():
    info = plsc.get_sparse_core_info()
    nw = info.num_cores * info.num_subcores
    bpw = _B // nw
    mesh = plsc.VectorSubcoreMesh(core_axis_name="c", subcore_axis_name="s")

    @functools.partial(
        pl.kernel, mesh=mesh,
        out_type=jax.ShapeDtypeStruct((_B, 2 * _D), jnp.float32),
        scratch_types=[
            pltpu.VMEM((bpw,), jnp.int32),
            pltpu.VMEM((bpw, 2 * _D), jnp.float32),
            pltpu.SemaphoreType.DMA,
        ],
    )
    def gather(table_hbm, idx_hbm, out_hbm, idx_v, rows_v, sem):
        wid = lax.axis_index("s") * info.num_cores + lax.axis_index("c")
        base = wid * bpw
        pltpu.sync_copy(idx_hbm.at[pl.ds(base, bpw)], idx_v)
        pltpu.async_copy(table_hbm.at[idx_v], rows_v, sem).wait()
        pltpu.sync_copy(rows_v, out_hbm.at[pl.ds(base, bpw)])

    return gather


def kernel(indices, embeddings, u):
    idx = indices.astype(jnp.int32)
    parity = (idx & 1).astype(jnp.float32).reshape(_B, 1)
    e2m = embeddings.reshape(_N // 2, 2 * _D)
    u1 = u.reshape(_N)
    ue = u1[0::2].reshape(_NSTEP, 1, _R)
    uo = u1[1::2].reshape(_NSTEP, 1, _R)
    inv_sigma = _sigma_pass(e2m, ue, uo)
    idx2 = (idx >> 1) + jnp.int32(inv_sigma[0, 0] * 0.0)
    rows2 = _make_gather()(e2m, idx2)
    return _scale(inv_sigma, parity, rows2)
